# baseline (speedup 1.0000x reference)
import functools

import jax
import jax.numpy as jnp
from jax import lax
from jax.experimental import pallas as pl
from jax.experimental.pallas import tpu as pltpu
from jax.experimental.pallas import tpu_sc as plsc

BATCH = 16384
NC = 2
NS = 16
NW = NC * NS
RPW = BATCH // NW
CHUNK = 128
NCH = RPW // CHUNK
LANES = 16
GW = 512
GPW = 61
COLS_PW = GPW * GW
TAIL0 = 1953 * GW
NV = 1000000
HITCAP = BATCH + LANES
NB = 8
FLB = 128

_mesh = plsc.VectorSubcoreMesh(
    core_axis_name="c", subcore_axis_name="s", num_cores=NC, num_subcores=NS
)


@functools.partial(
    pl.kernel,
    out_type=(
        jax.ShapeDtypeStruct((BATCH + LANES, 128), jnp.float32),
        jax.ShapeDtypeStruct((BATCH + LANES, 128), jnp.float32),
    ),
    mesh=_mesh,
    scratch_types=(
        pltpu.VMEM((BATCH,), jnp.int32),
        pltpu.VMEM((HITCAP,), jnp.int32),
        pltpu.VMEM((HITCAP,), jnp.int32),
        pltpu.VMEM((HITCAP,), jnp.int32),
        pltpu.VMEM((HITCAP,), jnp.int32),
        pltpu.VMEM((LANES,), jnp.int32),
        pltpu.VMEM((LANES,), jnp.int32),
        pltpu.VMEM((LANES,), jnp.int32),
        (pltpu.VMEM((LANES, GW), jnp.float32),) * 2,
        pltpu.VMEM((FLB, 128), jnp.float32),
        pltpu.VMEM((FLB,), jnp.int32),
        pltpu.SemaphoreType.DMA,
        pltpu.SemaphoreType.DMA,
        pltpu.SemaphoreType.DMA,
    ),
    compiler_params=pltpu.CompilerParams(needs_layout_passes=False),
)
def _extract(
    ueT_hbm, meT_hbm, uidx_hbm, midx_hbm,
    ug_hbm, mg_hbm,
    idx_v, hitr_v, hitp_v, slr_v, slp_v, cst_v, offs_v, ends_v, slabs,
    stv_v, sti_v, sem0, sem1, semsc,
):
    wid = lax.axis_index("s") * NC + lax.axis_index("c")
    lo = wid * COLS_PW
    hi = jnp.where(wid == NW - 1, NV, lo + COLS_PW)
    rows = lax.iota(jnp.int32, LANES)
    sems = (sem0, sem1)

    def splat(x):
        return jnp.full((LANES,), x, jnp.int32)

    def scan_hits():
        def chunk(t, cur):
            sel = splat(t * LANES) + rows
            rv = plsc.load_gather(idx_v, [sel])
            m = (rv >= lo) & (rv < hi)
            cnt = plsc.all_reduce_population_count(m)[0]
            plsc.store_compressed(cst_v.at[:], rv, mask=m)
            plsc.store_scatter(hitr_v, [splat(cur) + rows], cst_v[...])
            plsc.store_compressed(cst_v.at[:], sel, mask=m)
            plsc.store_scatter(hitp_v, [splat(cur) + rows], cst_v[...])
            return cur + cnt

        n = lax.fori_loop(0, BATCH // LANES, chunk, 0)
        plsc.store_scatter(hitr_v, [splat(n) + rows], splat(jnp.int32(1 << 30)))
        return n

    def bucket_sort(n):
        nch = (n + LANES - 1) // LANES

        def cnt_chunk(t, cs):
            sel = splat(t * LANES) + rows
            rv = plsc.load_gather(hitr_v, [sel])
            cb = (rv - lo) >> 12
            return tuple(
                cs[b] + plsc.all_reduce_population_count(cb == b)[0]
                for b in range(NB)
            )

        cnts = lax.fori_loop(0, nch, cnt_chunk, (jnp.int32(0),) * NB)
        off = jnp.int32(0)
        for b in range(NB):
            plsc.store_scatter(offs_v, [splat(b)], splat(off), mask=(rows == 0))
            plsc.store_scatter(
                ends_v, [splat(b)], splat(off + cnts[b]), mask=(rows == 0)
            )

            def app_chunk(t, cur, b=b):
                sel = splat(t * LANES) + rows
                rv = plsc.load_gather(hitr_v, [sel])
                pv = plsc.load_gather(hitp_v, [sel])
                m = ((rv - lo) >> 12) == b
                cnt = plsc.all_reduce_population_count(m)[0]
                plsc.store_compressed(cst_v.at[:], rv, mask=m)
                plsc.store_scatter(slr_v, [splat(cur) + rows], cst_v[...])
                plsc.store_compressed(cst_v.at[:], pv, mask=m)
                plsc.store_scatter(slp_v, [splat(cur) + rows], cst_v[...])
                return cur + cnt

            lax.fori_loop(0, nch, app_chunk, off)
            off = off + cnts[b]
        plsc.store_scatter(slr_v, [splat(n) + rows], splat(jnp.int32(1 << 30)))

    def reset_sti():
        for c in range(FLB // LANES):
            plsc.store_scatter(sti_v, [splat(c * LANES) + rows], splat(BATCH))

    def make_process_slab(gdst_hbm):
        def flush(wkk):
            pltpu.async_copy(stv_v, gdst_hbm.at[sti_v], semsc).wait()
            reset_sti()
            return jnp.int32(0)

        def process_slab(bank, col0, width, wkk0):
            bg = (col0 - lo) >> 12
            start = plsc.load_gather(offs_v, [splat(bg)])[0]
            end = plsc.load_gather(ends_v, [splat(bg)])[0]

            def compact(t, cur):
                sel = splat(t * LANES) + rows
                inb = (sel >= start) & (sel < end)
                rv = plsc.load_gather(slr_v, [sel])
                pv = plsc.load_gather(slp_v, [sel])
                m = inb & (rv >= col0) & (rv < col0 + width)
                cnt = plsc.all_reduce_population_count(m)[0]
                plsc.store_compressed(cst_v.at[:], rv, mask=m)
                plsc.store_scatter(hitr_v, [splat(cur) + rows], cst_v[...])
                plsc.store_compressed(cst_v.at[:], pv, mask=m)
                plsc.store_scatter(hitp_v, [splat(cur) + rows], cst_v[...])
                return cur + cnt

            nloc = lax.fori_loop(
                start >> 4, (end + LANES - 1) >> 4, compact, jnp.int32(0)
            )

            def echunk(c, wkk):
                wkk = lax.cond(
                    wkk + LANES >= FLB, flush, lambda w: w, wkk
                )
                sel = splat(c * LANES) + rows
                valid = sel < nloc
                rv = plsc.load_gather(hitr_v, [sel])
                pv = plsc.load_gather(hitp_v, [sel])
                lanev = jnp.where(valid, rv - col0, 0)
                tgt = jnp.where(valid, splat(wkk) + rows, FLB - 1)
                posv = jnp.where(valid, pv, splat(BATCH))
                for e in range(LANES):
                    comp = plsc.load_gather(slabs[bank], [splat(e), lanev])
                    plsc.store_scatter(stv_v, [tgt, splat(e)], comp)
                plsc.store_scatter(sti_v, [tgt], posv)
                rem = jnp.minimum(nloc - c * LANES, LANES)
                return wkk + rem

            return lax.fori_loop(0, (nloc + LANES - 1) >> 4, echunk, wkk0)

        return process_slab, flush

    def run_table(tbl_hbm, idxsrc_hbm, gdst_hbm):
        process_slab, flush = make_process_slab(gdst_hbm)
        pltpu.sync_copy(idxsrc_hbm, idx_v)
        n = scan_hits()
        bucket_sort(n)
        reset_sti()

        c0 = pl.multiple_of(lo, 128)
        pltpu.async_copy(tbl_hbm.at[:, pl.ds(c0, GW)], slabs[0], sems[0])
        c1 = pl.multiple_of(lo + GW, 128)
        pltpu.async_copy(tbl_hbm.at[:, pl.ds(c1, GW)], slabs[1], sems[1])

        gcnt = GPW + jnp.where(wid == NW - 1, 1, 0)

        def super_body(h, wkk):
            for b in range(2):
                g = h * 2 + b

                @pl.when(g < gcnt)
                def _():
                    pltpu.make_async_copy(
                        tbl_hbm.at[:, pl.ds(0, GW)], slabs[b], sems[b]
                    ).wait()

                col0 = lo + g * GW
                wkk = lax.cond(
                    g < gcnt,
                    lambda w: process_slab(b, col0, GW, w),
                    lambda w: w,
                    wkk,
                )

                @pl.when(g + 2 < gcnt)
                def _():
                    cn = pl.multiple_of(lo + (g + 2) * GW, 128)
                    pltpu.async_copy(tbl_hbm.at[:, pl.ds(cn, GW)], slabs[b], sems[b])

            return wkk

        nsup = (GPW + 2) // 2
        wkk = lax.fori_loop(0, nsup, super_body, jnp.int32(0))

        @pl.when(wid == NW - 1)
        def _():
            pltpu.async_copy(
                tbl_hbm.at[:, pl.ds(pl.multiple_of(TAIL0, 128), 128)],
                slabs[0].at[:, pl.ds(0, 128)],
                sems[0],
            ).wait()

        wkk = lax.cond(
            wid == NW - 1,
            lambda w: process_slab(0, TAIL0, 128, w),
            lambda w: w,
            wkk,
        )

        lax.cond(wkk > 0, flush, lambda w: w, wkk)

    run_table(ueT_hbm, uidx_hbm, ug_hbm)
    run_table(meT_hbm, midx_hbm, mg_hbm)


@functools.partial(
    pl.kernel,
    out_type=(
        jax.ShapeDtypeStruct((NW * 128,), jnp.float32),
        jax.ShapeDtypeStruct((BATCH,), jnp.float32),
    ),
    mesh=_mesh,
    scratch_types=(
        pltpu.VMEM((NCH, CHUNK), jnp.int32),
        pltpu.VMEM((NCH, CHUNK), jnp.int32),
        pltpu.VMEM((RPW,), jnp.float32),
        pltpu.VMEM((RPW,), jnp.float32),
        pltpu.VMEM((CHUNK, 128), jnp.float32),
        pltpu.VMEM((CHUNK, 128), jnp.float32),
        pltpu.VMEM((128,), jnp.float32),
        pltpu.VMEM((RPW,), jnp.float32),
        pltpu.SemaphoreType.DMA,
    ),
    compiler_params=pltpu.CompilerParams(
        use_tc_tiling_on_sc=False, needs_layout_passes=False
    ),
)
def _dot_bias(
    ug_hbm, mg_hbm, uidx_hbm, midx_hbm, ub_hbm, mb_hbm,
    partials_hbm, bsum_hbm,
    uidx_v, midx_v, ub_v, mb_v, us_v, ms_v, stage_v, bs_v, sem,
):
    wid = lax.axis_index("s") * NC + lax.axis_index("c")
    rows = lax.iota(jnp.int32, LANES)

    idx_cps = []
    for c in range(NCH):
        idx_cps.append(pltpu.async_copy(uidx_hbm.at[wid * NCH + c], uidx_v.at[c], sem))
        idx_cps.append(pltpu.async_copy(midx_hbm.at[wid * NCH + c], midx_v.at[c], sem))
    for cp in idx_cps:
        cp.wait()

    cps = []
    for c in range(NCH):
        sl = pl.ds(c * CHUNK, CHUNK)
        cps.append(pltpu.async_copy(ub_hbm.at[uidx_v.at[c]], ub_v.at[sl], sem))
        cps.append(pltpu.async_copy(mb_hbm.at[midx_v.at[c]], mb_v.at[sl], sem))
    for cp in cps:
        cp.wait()

    acc = jnp.zeros((LANES,), jnp.float32)
    for c in range(NCH):
        base = wid * RPW + c * CHUNK
        cpu = pltpu.async_copy(ug_hbm.at[pl.ds(base, CHUNK)], us_v, sem)
        cpm = pltpu.async_copy(mg_hbm.at[pl.ds(base, CHUNK)], ms_v, sem)
        cpu.wait()
        cpm.wait()

        def dot_body(j, a):
            sl = pl.ds(0, LANES)
            return a + us_v[j, sl] * ms_v[j, sl]

        acc = lax.fori_loop(0, CHUNK, dot_body, acc, unroll=8)

    plsc.store_scatter(stage_v, [rows], acc)
    pltpu.sync_copy(stage_v, partials_hbm.at[pl.ds(wid * 128, 128)])

    def bias_body(k, carry):
        sl = pl.ds(k * LANES, LANES)
        bs_v[sl] = ub_v[sl] + mb_v[sl]
        return carry

    lax.fori_loop(0, RPW // LANES, bias_body, 0, unroll=4)
    pltpu.sync_copy(bs_v, bsum_hbm.at[pl.ds(wid * RPW, RPW)])


@functools.partial(
    pl.kernel,
    out_type=jax.ShapeDtypeStruct((BATCH,), jnp.float32),
    mesh=_mesh,
    scratch_types=(
        pltpu.VMEM((NW * 128,), jnp.float32),
        pltpu.VMEM((RPW,), jnp.float32),
        pltpu.VMEM((RPW,), jnp.float32),
    ),
    compiler_params=pltpu.CompilerParams(
        use_tc_tiling_on_sc=False, needs_layout_passes=False
    ),
)
def _sigmoid_fin(partials_hbm, bsum_hbm, out_hbm, part_v, b_v, o_v):
    wid = lax.axis_index("s") * NC + lax.axis_index("c")
    pltpu.sync_copy(partials_hbm, part_v)
    pltpu.sync_copy(bsum_hbm.at[pl.ds(wid * RPW, RPW)], b_v)

    acc = part_v[pl.ds(0, LANES)]
    for w in range(1, NW):
        acc = acc + part_v[pl.ds(w * 128, LANES)]
    s = jnp.sum(acc)

    def sig_body(k, carry):
        sl = pl.ds(k * LANES, LANES)
        x = s + b_v[sl]
        o_v[sl] = 1.0 / (1.0 + jnp.exp(-x))
        return carry

    lax.fori_loop(0, RPW // LANES, sig_body, 0, unroll=4)
    pltpu.sync_copy(o_v, out_hbm.at[pl.ds(wid * RPW, RPW)])


def kernel(inputs, user_embedding, movie_embedding, user_bias, movie_bias):
    uidx = inputs[:, 0]
    midx = inputs[:, 1]
    uidx2 = uidx.reshape(NW * NCH, CHUNK)
    midx2 = midx.reshape(NW * NCH, CHUNK)
    ub = user_bias.reshape(-1)
    mb = movie_bias.reshape(-1)
    ug, mg = _extract(user_embedding.T, movie_embedding.T, uidx, midx)
    partials, bsum = _dot_bias(ug, mg, uidx2, midx2, ub, mb)
    out = _sigmoid_fin(partials, bsum)
    return out.reshape(BATCH, 1)


# speedup vs baseline: 3.4905x; 3.4905x over previous
import functools

import jax
import jax.numpy as jnp
from jax import lax
from jax.experimental import pallas as pl
from jax.experimental.pallas import tpu as pltpu
from jax.experimental.pallas import tpu_sc as plsc

BATCH = 16384
EMBED = 16
NC = 2
NS = 16
NW = NC * NS
RPW = BATCH // NW
CHUNK = 128
NCH = RPW // CHUNK
LANES = 16
GRP = 8
NSUP = RPW // (2 * GRP)

_mesh = plsc.VectorSubcoreMesh(
    core_axis_name="c", subcore_axis_name="s", num_cores=NC, num_subcores=NS
)

_slab = pltpu.VMEM((LANES, 128), jnp.float32)


@functools.partial(
    pl.kernel,
    out_type=jax.ShapeDtypeStruct((NW * 128,), jnp.float32),
    mesh=_mesh,
    scratch_types=(
        pltpu.VMEM((RPW + LANES,), jnp.int32),
        pltpu.VMEM((RPW + LANES,), jnp.int32),
        ((_slab,) * GRP, (_slab,) * GRP),
        ((_slab,) * GRP, (_slab,) * GRP),
        pltpu.VMEM((128,), jnp.float32),
        ((pltpu.SemaphoreType.DMA,) * GRP,) * 2,
    ),
    compiler_params=pltpu.CompilerParams(needs_layout_passes=False),
)
def _dot_partial(
    ueT_hbm, meT_hbm, uidx_hbm, midx_hbm,
    partials_hbm,
    uidx_v, midx_v, uslabs, mslabs, stage_v, sems,
):
    wid = lax.axis_index("s") * NC + lax.axis_index("c")
    base = pl.multiple_of(wid * RPW, 128)
    pltpu.sync_copy(uidx_hbm.at[pl.ds(base, RPW)], uidx_v.at[pl.ds(0, RPW)])
    pltpu.sync_copy(midx_hbm.at[pl.ds(base, RPW)], midx_v.at[pl.ds(0, RPW)])

    rows = lax.iota(jnp.int32, LANES)

    def idx_vecs(jbase):
        sel = rows + jnp.full((LANES,), jbase, jnp.int32)
        uvec = plsc.load_gather(uidx_v, [sel])
        mvec = plsc.load_gather(midx_v, [sel])
        return uvec, mvec

    def fire(bank, jbase):
        uvec, mvec = idx_vecs(jbase)
        for b in range(GRP):
            offu = pl.multiple_of(((uvec[b] >> 7) * 128).astype(jnp.int32), 128)
            offm = pl.multiple_of(((mvec[b] >> 7) * 128).astype(jnp.int32), 128)
            pltpu.async_copy(
                ueT_hbm.at[:, pl.ds(offu, 128)], uslabs[bank][b], sems[bank][b]
            )
            pltpu.async_copy(
                meT_hbm.at[:, pl.ds(offm, 128)], mslabs[bank][b], sems[bank][b]
            )

    def consume(bank, jbase, acc):
        uvec, mvec = idx_vecs(jbase)
        ulane = uvec & 127
        mlane = mvec & 127
        for b in range(GRP):
            pltpu.make_async_copy(
                ueT_hbm.at[:, pl.ds(0, 128)], uslabs[bank][b], sems[bank][b]
            ).wait()
            pltpu.make_async_copy(
                meT_hbm.at[:, pl.ds(0, 128)], mslabs[bank][b], sems[bank][b]
            ).wait()
            lu = jnp.full((LANES,), ulane[b], jnp.int32)
            lm = jnp.full((LANES,), mlane[b], jnp.int32)
            ucol = plsc.load_gather(uslabs[bank][b], [rows, lu])
            mcol = plsc.load_gather(mslabs[bank][b], [rows, lm])
            acc = acc + ucol * mcol
        return acc

    fire(0, 0)
    fire(1, GRP)

    def super_body(h, acc):
        jb = h * 2 * GRP
        acc = consume(0, jb, acc)

        @pl.when(h < NSUP - 1)
        def _():
            fire(0, jb + 2 * GRP)

        acc = consume(1, jb + GRP, acc)

        @pl.when(h < NSUP - 1)
        def _():
            fire(1, jb + 3 * GRP)

        return acc

    acc = lax.fori_loop(0, NSUP, super_body, jnp.zeros((LANES,), jnp.float32))
    plsc.store_scatter(stage_v, [rows], acc)
    pltpu.sync_copy(
        stage_v, partials_hbm.at[pl.ds(pl.multiple_of(wid * 128, 128), 128)]
    )


@functools.partial(
    pl.kernel,
    out_type=jax.ShapeDtypeStruct((BATCH,), jnp.float32),
    mesh=_mesh,
    scratch_types=(
        pltpu.VMEM((NCH, CHUNK), jnp.int32),
        pltpu.VMEM((NCH, CHUNK), jnp.int32),
        pltpu.VMEM((RPW,), jnp.float32),
        pltpu.VMEM((RPW,), jnp.float32),
        pltpu.VMEM((NW * 128,), jnp.float32),
        pltpu.VMEM((RPW,), jnp.float32),
        pltpu.SemaphoreType.DMA,
    ),
    compiler_params=pltpu.CompilerParams(
        use_tc_tiling_on_sc=False, needs_layout_passes=False
    ),
)
def _finalize(
    uidx_hbm, midx_hbm, ub_hbm, mb_hbm, partials_hbm,
    out_hbm,
    uidx_v, midx_v, ub_v, mb_v, part_v, o_v, sem,
):
    wid = lax.axis_index("s") * NC + lax.axis_index("c")

    idx_cps = []
    for c in range(NCH):
        idx_cps.append(pltpu.async_copy(uidx_hbm.at[wid * NCH + c], uidx_v.at[c], sem))
        idx_cps.append(pltpu.async_copy(midx_hbm.at[wid * NCH + c], midx_v.at[c], sem))
    idx_cps.append(pltpu.async_copy(partials_hbm, part_v, sem))
    for cp in idx_cps:
        cp.wait()

    cps = []
    for c in range(NCH):
        sl = pl.ds(c * CHUNK, CHUNK)
        cps.append(pltpu.async_copy(ub_hbm.at[uidx_v.at[c]], ub_v.at[sl], sem))
        cps.append(pltpu.async_copy(mb_hbm.at[midx_v.at[c]], mb_v.at[sl], sem))
    for cp in cps:
        cp.wait()

    acc = part_v[pl.ds(0, LANES)]
    for w in range(1, NW):
        acc = acc + part_v[pl.ds(w * 128, LANES)]
    s = jnp.sum(acc)

    def sig_body(k, carry):
        sl = pl.ds(k * LANES, LANES)
        x = s + ub_v[sl] + mb_v[sl]
        o_v[sl] = 1.0 / (1.0 + jnp.exp(-x))
        return carry

    lax.fori_loop(0, RPW // LANES, sig_body, 0, unroll=4)
    pltpu.sync_copy(o_v, out_hbm.at[pl.ds(wid * RPW, RPW)])


def kernel(inputs, user_embedding, movie_embedding, user_bias, movie_bias):
    uidx = inputs[:, 0]
    midx = inputs[:, 1]
    uidx2 = uidx.reshape(NW * NCH, CHUNK)
    midx2 = midx.reshape(NW * NCH, CHUNK)
    ub = user_bias.reshape(-1)
    mb = movie_bias.reshape(-1)
    partials = _dot_partial(user_embedding.T, movie_embedding.T, uidx, midx)
    out = _finalize(uidx2, midx2, ub, mb, partials)
    return out.reshape(BATCH, 1)
